# Initial kernel scaffold; baseline (speedup 1.0000x reference)
import jax
import jax.numpy as jnp
from jax.experimental import pallas as pl


def kernel(x, W_enc, b_enc, W_dec, b_dec):
    raise NotImplementedError("write your pallas kernel here")


# speedup vs baseline: 14.1586x; 14.1586x over previous
import functools

import jax
import jax.numpy as jnp
from jax import lax
from jax.experimental import pallas as pl
from jax.experimental.pallas import tpu as pltpu

KTOP = 64
T_TOK = 2048
D_MODEL = 1024
D_SAE = 16384

ENC_CH = 1024
DEC_CH = 1024
THR_TB = 128
MAX_FINITE_BITS = 0x7F7FFFFF


def _encode_body(x_ref, w_ref, b_ref, z_ref):
    acc = jnp.dot(x_ref[...], w_ref[...], preferred_element_type=jnp.float32)
    z_ref[...] = jnp.maximum(acc + b_ref[...], 0.0)


def _threshold_body(z_ref, thr_ref):
    zb = z_ref[...]

    def step(_, carry):
        lo, hi = carry
        mid = lo + ((hi - lo + 1) >> 1)
        mid_f = lax.bitcast_convert_type(mid, jnp.float32)
        cnt = jnp.sum((zb >= mid_f).astype(jnp.int32), axis=1, keepdims=True)
        take = cnt >= KTOP
        lo = jnp.where(take, mid, lo)
        hi = jnp.where(take, hi, mid - 1)
        return lo, hi

    lo0 = jnp.zeros((THR_TB, 1), jnp.int32)
    hi0 = jnp.full((THR_TB, 1), MAX_FINITE_BITS, jnp.int32)
    lo, _ = lax.fori_loop(0, 31, step, (lo0, hi0))
    thr_ref[...] = jnp.broadcast_to(lo, (THR_TB, 128))


def _decode_body(z_ref, thr_ref, w_ref, b_ref, out_ref):
    c = pl.program_id(0)
    thr = lax.bitcast_convert_type(thr_ref[:, 0:1], jnp.float32)
    zb = z_ref[...]
    zs = jnp.where(zb >= thr, zb, 0.0)
    partial = jnp.dot(zs, w_ref[...], preferred_element_type=jnp.float32)

    @pl.when(c == 0)
    def _():
        out_ref[...] = partial + b_ref[...]

    @pl.when(c != 0)
    def _():
        out_ref[...] += partial


def kernel(x, W_enc, b_enc, W_dec, b_dec):
    n_enc = D_SAE // ENC_CH
    z = pl.pallas_call(
        _encode_body,
        grid=(n_enc,),
        in_specs=[
            pl.BlockSpec((T_TOK, D_MODEL), lambda c: (0, 0)),
            pl.BlockSpec((D_MODEL, ENC_CH), lambda c: (0, c)),
            pl.BlockSpec((1, ENC_CH), lambda c: (0, c)),
        ],
        out_specs=pl.BlockSpec((T_TOK, ENC_CH), lambda c: (0, c)),
        out_shape=jax.ShapeDtypeStruct((T_TOK, D_SAE), jnp.float32),
        compiler_params=pltpu.CompilerParams(
            dimension_semantics=("arbitrary",)),
    )(x, W_enc, b_enc.reshape(1, D_SAE))

    n_tb = T_TOK // THR_TB
    thr = pl.pallas_call(
        _threshold_body,
        grid=(n_tb,),
        in_specs=[pl.BlockSpec((THR_TB, D_SAE), lambda t: (t, 0))],
        out_specs=pl.BlockSpec((THR_TB, 128), lambda t: (t, 0)),
        out_shape=jax.ShapeDtypeStruct((T_TOK, 128), jnp.int32),
        compiler_params=pltpu.CompilerParams(
            dimension_semantics=("arbitrary",)),
    )(z)

    n_dec = D_SAE // DEC_CH
    out = pl.pallas_call(
        _decode_body,
        grid=(n_dec,),
        in_specs=[
            pl.BlockSpec((T_TOK, DEC_CH), lambda c: (0, c)),
            pl.BlockSpec((T_TOK, 128), lambda c: (0, 0)),
            pl.BlockSpec((DEC_CH, D_MODEL), lambda c: (c, 0)),
            pl.BlockSpec((1, D_MODEL), lambda c: (0, 0)),
        ],
        out_specs=pl.BlockSpec((T_TOK, D_MODEL), lambda c: (0, 0)),
        out_shape=jax.ShapeDtypeStruct((T_TOK, D_MODEL), jnp.float32),
        compiler_params=pltpu.CompilerParams(
            dimension_semantics=("arbitrary",)),
    )(z, thr, W_dec, b_dec.reshape(1, D_MODEL))
    return out
